# baseline (speedup 1.0000x reference)
import functools

import jax
import jax.numpy as jnp
from jax import lax
from jax.experimental import pallas as pl
from jax.experimental.pallas import tpu as pltpu
from jax.experimental.pallas import tpu_sc as plsc

_NC = 2
_NS = 16
_NW = _NC * _NS


def _embed_sums_sc(ctx_t, emb_table, scale):
    L, B = ctx_t.shape
    V, D = emb_table.shape
    bw = B // _NW
    nd = D // 16

    mesh = plsc.VectorSubcoreMesh(core_axis_name="c", subcore_axis_name="s")

    @functools.partial(
        pl.kernel,
        out_type=jax.ShapeDtypeStruct((B, D), jnp.float32),
        mesh=mesh,
        scratch_types=[
            pltpu.VMEM((L, bw), jnp.int32),
            pltpu.VMEM((bw, D), jnp.float32),
            pltpu.VMEM((bw, D), jnp.float32),
            pltpu.VMEM((bw, D), jnp.float32),
            pltpu.SemaphoreType.DMA,
            pltpu.SemaphoreType.DMA,
        ],
        compiler_params=pltpu.CompilerParams(use_tc_tiling_on_sc=False),
    )
    def sc_kernel(ctx_hbm, emb_hbm, out_hbm, idx_v, buf0, buf1, acc, sem0, sem1):
        wid = lax.axis_index("s") * _NC + lax.axis_index("c")
        base = wid * bw
        pltpu.sync_copy(ctx_hbm.at[:, pl.ds(base, bw)], idx_v)
        bufs = (buf0, buf1)
        sems = (sem0, sem1)
        copies = [None, None]
        copies[0] = pltpu.async_copy(emb_hbm.at[idx_v.at[0]], buf0, sem0)
        for j in range(L):
            if j + 1 < L:
                nxt = (j + 1) % 2
                copies[nxt] = pltpu.async_copy(
                    emb_hbm.at[idx_v.at[j + 1]], bufs[nxt], sems[nxt])
            copies[j % 2].wait()
            buf = bufs[j % 2]
            if j == 0:
                def body(r, c, buf=buf):
                    for d in range(nd):
                        acc[r, pl.ds(d * 16, 16)] = buf[r, pl.ds(d * 16, 16)]
                    return c
            elif j < L - 1:
                def body(r, c, buf=buf):
                    for d in range(nd):
                        acc[r, pl.ds(d * 16, 16)] = (
                            acc[r, pl.ds(d * 16, 16)] + buf[r, pl.ds(d * 16, 16)])
                    return c
            else:
                def body(r, c, buf=buf):
                    for d in range(nd):
                        acc[r, pl.ds(d * 16, 16)] = (
                            acc[r, pl.ds(d * 16, 16)] + buf[r, pl.ds(d * 16, 16)]
                        ) * scale
                    return c
            lax.fori_loop(0, bw, body, 0, unroll=4)
        pltpu.sync_copy(acc, out_hbm.at[pl.ds(base, bw), :])

    return sc_kernel(ctx_t, emb_table)


def _linear_tc(sumsT, WT, b2, half, out_prev):
    D, BH = sumsT.shape
    V = WT.shape[1]
    B = 2 * BH
    BN = 512
    nv = pl.cdiv(V, BN)

    def mm(*refs):
        e_ref, w_ref, b_ref = refs[0], refs[1], refs[2]
        o_ref = refs[-1]
        o_ref[...] = lax.dot_general(
            w_ref[...], e_ref[...], (((0,), (0,)), ((), ())),
            preferred_element_type=jnp.float32) + b_ref[...]

    in_specs = [
        pl.BlockSpec((D, BH), lambda i: (0, 0)),
        pl.BlockSpec((D, BN), lambda i: (0, i)),
        pl.BlockSpec((BN, 1), lambda i: (i, 0)),
    ]
    args = [sumsT, WT, b2]
    aliases = {}
    if out_prev is not None:
        in_specs.append(pl.BlockSpec(memory_space=pltpu.MemorySpace.HBM))
        args.append(out_prev)
        aliases = {3: 0}

    return pl.pallas_call(
        mm,
        grid=(nv,),
        in_specs=in_specs,
        out_specs=pl.BlockSpec((BN, BH), lambda i: (i, half)),
        out_shape=jax.ShapeDtypeStruct((V, B), jnp.float32),
        input_output_aliases=aliases,
    )(*args)


def kernel(context, emb_table, W, b):
    B, L = context.shape
    ctx_t = context.astype(jnp.int32).T
    WT = W.T
    b2 = b.reshape(-1, 1)
    half = B // 2
    sums0 = _embed_sums_sc(ctx_t[:, :half], emb_table, 1.0 / L)
    sums1 = _embed_sums_sc(ctx_t[:, half:], emb_table, 1.0 / L)
    out0 = _linear_tc(sums0.T, WT, b2, 0, None)
    out1 = _linear_tc(sums1.T, WT, b2, 1, out0)
    return out1.T


# speedup vs baseline: 1.2195x; 1.2195x over previous
import functools

import jax
import jax.numpy as jnp
from jax import lax
from jax.experimental import pallas as pl
from jax.experimental.pallas import tpu as pltpu
from jax.experimental.pallas import tpu_sc as plsc

_NC = 2
_NS = 16
_NW = _NC * _NS
_NBUF = 4


def _embed_sums_sc(ctx_t, emb_table, scale):
    L, B = ctx_t.shape
    V, D = emb_table.shape
    bw = B // _NW
    nd = D // 16

    mesh = plsc.VectorSubcoreMesh(core_axis_name="c", subcore_axis_name="s")

    @functools.partial(
        pl.kernel,
        out_type=jax.ShapeDtypeStruct((B, D), jnp.float32),
        mesh=mesh,
        scratch_types=[
            pltpu.VMEM((L, bw), jnp.int32),
            pltpu.VMEM((bw, D), jnp.float32),
        ]
        + [pltpu.VMEM((bw, D), jnp.float32) for _ in range(_NBUF)]
        + [pltpu.SemaphoreType.DMA for _ in range(_NBUF)],
        compiler_params=pltpu.CompilerParams(use_tc_tiling_on_sc=False),
    )
    def sc_kernel(ctx_hbm, emb_hbm, out_hbm, idx_v, acc, *bufsem):
        bufs = bufsem[:_NBUF]
        sems = bufsem[_NBUF:]
        wid = lax.axis_index("s") * _NC + lax.axis_index("c")
        base = wid * bw
        pltpu.sync_copy(ctx_hbm.at[:, pl.ds(base, bw)], idx_v)
        copies = [None] * _NBUF
        for j in range(min(_NBUF - 1, L)):
            copies[j] = pltpu.async_copy(
                emb_hbm.at[idx_v.at[j]], bufs[j], sems[j])
        for j in range(L):
            nxt = j + _NBUF - 1
            if nxt < L:
                s = nxt % _NBUF
                copies[s] = pltpu.async_copy(
                    emb_hbm.at[idx_v.at[nxt]], bufs[s], sems[s])
            copies[j % _NBUF].wait()
            buf = bufs[j % _NBUF]
            if j == 0:
                def body(r, c, buf=buf):
                    for d in range(nd):
                        acc[r, pl.ds(d * 16, 16)] = buf[r, pl.ds(d * 16, 16)]
                    return c
            elif j < L - 1:
                def body(r, c, buf=buf):
                    for d in range(nd):
                        acc[r, pl.ds(d * 16, 16)] = (
                            acc[r, pl.ds(d * 16, 16)] + buf[r, pl.ds(d * 16, 16)])
                    return c
            else:
                def body(r, c, buf=buf):
                    for d in range(nd):
                        acc[r, pl.ds(d * 16, 16)] = (
                            acc[r, pl.ds(d * 16, 16)] + buf[r, pl.ds(d * 16, 16)]
                        ) * scale
                    return c
            lax.fori_loop(0, bw, body, 0, unroll=4)
        pltpu.sync_copy(acc, out_hbm.at[pl.ds(base, bw), :])

    return sc_kernel(ctx_t, emb_table)


def _linear_tc(sumsT, WT, b2):
    D, B = sumsT.shape
    V = WT.shape[1]
    BN = 512
    nv = pl.cdiv(V, BN)

    def mm(e_ref, w_ref, b_ref, o_ref):
        o_ref[...] = lax.dot_general(
            w_ref[...], e_ref[...], (((0,), (0,)), ((), ())),
            preferred_element_type=jnp.float32) + b_ref[...]

    return pl.pallas_call(
        mm,
        grid=(nv,),
        in_specs=[
            pl.BlockSpec((D, B), lambda i: (0, 0)),
            pl.BlockSpec((D, BN), lambda i: (0, i)),
            pl.BlockSpec((BN, 1), lambda i: (i, 0)),
        ],
        out_specs=pl.BlockSpec((BN, B), lambda i: (i, 0)),
        out_shape=jax.ShapeDtypeStruct((V, B), jnp.float32),
    )(sumsT, WT, b2)


def kernel(context, emb_table, W, b):
    B, L = context.shape
    ctx_t = context.astype(jnp.int32).T
    WT = W.T
    sums = _embed_sums_sc(ctx_t, emb_table, 1.0 / L)
    outT = _linear_tc(sums.T, WT, b.reshape(-1, 1))
    return outT.T


# speedup vs baseline: 1.2412x; 1.0178x over previous
import functools

import jax
import jax.numpy as jnp
from jax import lax
from jax.experimental import pallas as pl
from jax.experimental.pallas import tpu as pltpu
from jax.experimental.pallas import tpu_sc as plsc

_NC = 2
_NS = 16
_NW = _NC * _NS
_NBUF = 4


def _embed_sums_sc(ctx_t, emb_table, scale):
    L, B = ctx_t.shape
    V, D = emb_table.shape
    bw = B // _NW
    nd = D // 16

    mesh = plsc.VectorSubcoreMesh(core_axis_name="c", subcore_axis_name="s")
    RC = 32
    nch = bw // RC
    nset = 2

    @functools.partial(
        pl.kernel,
        out_type=jax.ShapeDtypeStruct((B, D), jnp.float32),
        mesh=mesh,
        scratch_types=[
            pltpu.VMEM((L, bw), jnp.int32),
            pltpu.VMEM((nset, RC, D), jnp.float32),
        ]
        + [pltpu.VMEM((nset, L, RC, D), jnp.float32)]
        + [pltpu.SemaphoreType.DMA for _ in range(nset)],
        compiler_params=pltpu.CompilerParams(use_tc_tiling_on_sc=False),
    )
    def sc_kernel(ctx_hbm, emb_hbm, out_hbm, idx_v, res, gbuf, *sems):
        wid = lax.axis_index("s") * _NC + lax.axis_index("c")
        base = wid * bw
        pltpu.sync_copy(ctx_hbm.at[:, pl.ds(base, bw)], idx_v)

        def fire(c):
            st = c % nset
            cps = []
            for j in range(L):
                cps.append(pltpu.async_copy(
                    emb_hbm.at[idx_v.at[j, pl.ds(c * RC, RC)]],
                    gbuf.at[st, j], sems[st]))
            return cps

        def drain(cps):
            for cp in cps:
                cp.wait()

        inflight = fire(0)
        for c in range(nch):
            if c + 1 < nch:
                nxt = fire(c + 1)
            drain(inflight)
            st = c % nset

            def body(r, carry, st=st, c=c):
                for d in range(nd):
                    v = gbuf[st, 0, r, pl.ds(d * 16, 16)]
                    for j in range(1, L):
                        v = v + gbuf[st, j, r, pl.ds(d * 16, 16)]
                    res[st, r, pl.ds(d * 16, 16)] = v * scale
                return carry

            lax.fori_loop(0, RC, body, 0, unroll=2)
            pltpu.sync_copy(
                res.at[st],
                out_hbm.at[pl.ds(base + c * RC, RC), :])
            if c + 1 < nch:
                inflight = nxt

    return sc_kernel(ctx_t, emb_table)


def _linear_tc(sumsT, WT, b2):
    D, B = sumsT.shape
    V = WT.shape[1]
    BN = 512
    nv = pl.cdiv(V, BN)

    def mm(e_ref, w_ref, b_ref, o_ref):
        o_ref[...] = lax.dot_general(
            w_ref[...], e_ref[...], (((0,), (0,)), ((), ())),
            preferred_element_type=jnp.float32) + b_ref[...]

    return pl.pallas_call(
        mm,
        grid=(nv,),
        in_specs=[
            pl.BlockSpec((D, B), lambda i: (0, 0)),
            pl.BlockSpec((D, BN), lambda i: (0, i)),
            pl.BlockSpec((BN, 1), lambda i: (i, 0)),
        ],
        out_specs=pl.BlockSpec((BN, B), lambda i: (i, 0)),
        out_shape=jax.ShapeDtypeStruct((V, B), jnp.float32),
    )(sumsT, WT, b2)


def kernel(context, emb_table, W, b):
    B, L = context.shape
    ctx_t = context.astype(jnp.int32).T
    WT = W.T
    sums = _embed_sums_sc(ctx_t, emb_table, 1.0 / L)
    outT = _linear_tc(sums.T, WT, b.reshape(-1, 1))
    return outT.T
